# Initial kernel scaffold; baseline (speedup 1.0000x reference)
import jax
import jax.numpy as jnp
from jax.experimental import pallas as pl


def kernel(h0, x0, dfeat, p_dfeat, edge_index, params):
    raise NotImplementedError("write your pallas kernel here")


# speedup vs baseline: 84.6396x; 84.6396x over previous
import functools
import math

import jax
import jax.numpy as jnp
import numpy as np
from jax import lax
from jax.experimental import pallas as pl
from jax.experimental.pallas import tpu as pltpu
from jax.experimental.pallas import tpu_sc as plsc

N = 10000
E = 160000
BE = 2000
GE = E // BE
BN = 1000
ROWS2 = E // 128
W_ROWS = 39
T_ROWS = 78
NC, NSUB = 2, 16

_SQ3 = math.sqrt(3.0)
_S20 = 1.0 / math.sqrt(20.0)
_S24 = 1.0 / math.sqrt(24.0)
_S48 = 1.0 / math.sqrt(48.0)

_EPS3 = np.zeros((3, 3, 3), np.float32)
for _i, _j, _k in [(0, 1, 2), (1, 2, 0), (2, 0, 1)]:
    _EPS3[_i, _j, _k] = 1.0
    _EPS3[_i, _k, _j] = -1.0


def _build_conv0():
    Kw = 320
    A1 = np.zeros((16, Kw), np.float32)
    C1 = np.zeros((Kw, 20), np.float32)
    for u in range(16):
        for v in range(16):
            A1[u, u * 16 + v] = 1
            C1[u * 16 + v, v] = 0.25
        for v in range(4):
            A1[u, 256 + u * 4 + v] = 1
            C1[256 + u * 4 + v, 16 + v] = 0.25
    od = 32
    PDIR = np.zeros((20, od), np.float32)
    PSH = np.zeros((20, od), np.float32)
    SSH = np.zeros((4, od), np.float32)
    for v in range(16):
        PDIR[v, v] = 1
    for v in range(4):
        for j in range(3):
            PSH[16 + v, 16 + v * 3 + j] = 1
            SSH[1 + j, 16 + v * 3 + j] = 1
    return dict(A0=None, B0=None, C0=None, wsplit=[(0, 320, None)],
                A1=A1, C1=C1, PDIR=PDIR, PSH=PSH, SSH=SSH, fw=16)


def _build_conv1():
    A0 = np.zeros((28, 36), np.float32)
    B0 = np.zeros((4, 36), np.float32)
    C0 = np.zeros((36, 16), np.float32)
    for u in range(4):
        for i in range(3):
            for j in range(3):
                c = u * 9 + i * 3 + j
                A0[16 + u * 3 + i, c] = 1
                B0[1 + j, c] = 1
                for k in range(3):
                    if _EPS3[i, j, k] != 0:
                        C0[c, 4 + u * 3 + k] += _EPS3[i, j, k]
            C0[u * 9 + i * 3 + i, u] = 1
    WDE = np.zeros((32, 96), np.float32)
    for t in range(16):
        for j in range(3):
            WDE[t, t * 3 + j] = 1
            WDE[16 + t, 48 + t * 3 + j] = 1
    Kw = 480
    A1 = np.zeros((44, Kw), np.float32)
    C1 = np.zeros((Kw, 44), np.float32)
    for u in range(16):
        for v in range(16):
            A1[u, u * 16 + v] = 1
            C1[u * 16 + v, v] = _S20
        for v in range(4):
            A1[u, 320 + u * 4 + v] = 1
            C1[320 + u * 4 + v, 16 + v] = _S20
    for u in range(4):
        for v in range(16):
            A1[28 + u, 256 + u * 16 + v] = 1
            C1[256 + u * 16 + v, v] = _S20 / _SQ3
        for v in range(4):
            for j in range(3):
                A1[16 + u * 3 + j, 384 + (u * 4 + v) * 3 + j] = 1
                C1[384 + (u * 4 + v) * 3 + j, 20 + v * 3 + j] = _S20
                A1[32 + u * 3 + j, 432 + (u * 4 + v) * 3 + j] = 1
                C1[432 + (u * 4 + v) * 3 + j, 32 + v * 3 + j] = 1.0 / (2.0 * math.sqrt(2.0))
    od = 48
    PDIR = np.zeros((44, od), np.float32)
    PSH = np.zeros((44, od), np.float32)
    SSH = np.zeros((4, od), np.float32)
    for v in range(16):
        PDIR[v, v] = 1
    for v in range(4):
        for j in range(3):
            PDIR[20 + v * 3 + j, 16 + v * 3 + j] = 1
            PSH[16 + v, 16 + v * 3 + j] = 1
            SSH[1 + j, 16 + v * 3 + j] = 1
            PDIR[32 + v * 3 + j, 28 + v * 3 + j] = 1
    return dict(A0=A0, B0=B0, C0=C0, wsplit=[(0, 384, None), (384, 416, WDE)],
                A1=A1, C1=C1, PDIR=PDIR, PSH=PSH, SSH=SSH, fw=28)


def _wq():
    WQ = np.zeros((4, 12), np.float32)
    for u in range(4):
        for j in range(3):
            WQ[u, u * 3 + j] = 1
    return WQ


def _build_coord0():
    Kw = 28
    A1 = np.zeros((28, Kw), np.float32)
    C1 = np.zeros((Kw, 4), np.float32)
    for u in range(16):
        A1[u, u] = 1
        C1[u, 0] = _S20
    for u in range(4):
        for j in range(3):
            A1[16 + u * 3 + j, 16 + u * 3 + j] = 1
            C1[16 + u * 3 + j, 1 + j] = _S20
    od = 16
    PDIR = np.zeros((4, od), np.float32)
    PSH = np.zeros((4, od), np.float32)
    SSH = np.zeros((4, od), np.float32)
    for j in range(3):
        PDIR[1 + j, j] = 1
        PSH[0, j] = 1
        SSH[1 + j, j] = 1
    return dict(A0=None, B0=None, C0=None, wsplit=[(0, 16, None), (16, 20, _wq())],
                A1=A1, C1=C1, PDIR=PDIR, PSH=PSH, SSH=SSH, fw=28)


def _build_coord1():
    pairs = [(i, j) for i in range(3) for j in range(3) if i != j]
    A0 = np.zeros((40, 24), np.float32)
    B0 = np.zeros((4, 24), np.float32)
    C0 = np.zeros((24, 12), np.float32)
    for u in range(4):
        for pi, (i, j) in enumerate(pairs):
            c = u * 6 + pi
            A0[28 + u * 3 + i, c] = 1
            B0[1 + j, c] = 1
            for k in range(3):
                if _EPS3[i, j, k] != 0:
                    C0[c, u * 3 + k] += _EPS3[i, j, k]
    Kw = 40
    A1 = np.zeros((52, Kw), np.float32)
    C1 = np.zeros((Kw, 7), np.float32)
    for u in range(16):
        A1[u, u] = 1
        C1[u, 0] = _S24
    for u in range(4):
        for j in range(3):
            A1[16 + u * 3 + j, 16 + u * 3 + j] = 1
            C1[16 + u * 3 + j, 1 + j] = _S24
            A1[40 + u * 3 + j, 28 + u * 3 + j] = 1
            C1[28 + u * 3 + j, 4 + j] = _S48
    od = 16
    PDIR = np.zeros((7, od), np.float32)
    PSH = np.zeros((7, od), np.float32)
    SSH = np.zeros((4, od), np.float32)
    for j in range(3):
        PDIR[1 + j, j] = 1
        PDIR[4 + j, j] = 1
        PSH[0, j] = 1
        SSH[1 + j, j] = 1
    return dict(A0=A0, B0=B0, C0=C0,
                wsplit=[(0, 16, None), (16, 20, _wq()), (20, 24, _wq())],
                A1=A1, C1=C1, PDIR=PDIR, PSH=PSH, SSH=SSH, fw=40)


_CONV0 = _build_conv0()
_CONV1 = _build_conv1()
_COORD0 = _build_coord0()
_COORD1 = _build_coord1()


def _tp_consts(M):
    out = []
    if M['A0'] is not None:
        out += [jnp.asarray(M['A0']), jnp.asarray(M['B0']), jnp.asarray(M['C0'])]
    for (_, _, mat) in M['wsplit']:
        if mat is not None:
            out.append(jnp.asarray(mat))
    out += [jnp.asarray(M['A1']), jnp.asarray(M['C1']), jnp.asarray(M['PDIR']),
            jnp.asarray(M['PSH']), jnp.asarray(M['SSH'])]
    return out


def _tp_apply(feat, sh, w, M, consts):
    it = iter(consts)
    if M['A0'] is not None:
        A0, B0, C0 = next(it), next(it), next(it)
        aux = ((feat @ A0) * (sh @ B0)) @ C0
        feat2 = jnp.concatenate([feat, aux], axis=1)
    else:
        feat2 = feat
    parts = []
    for (a, b, mat) in M['wsplit']:
        if mat is None:
            parts.append(w[:, a:b])
        else:
            parts.append(w[:, a:b] @ next(it))
    wext = jnp.concatenate(parts, axis=1) if len(parts) > 1 else parts[0]
    A1, C1, PDIR, PSH, SSH = next(it), next(it), next(it), next(it), next(it)
    part = ((feat2 @ A1) * wext) @ C1
    return part @ PDIR + (part @ PSH) * (sh @ SSH)


def _elu(x):
    return jnp.where(x > 0, x, jnp.exp(jnp.minimum(x, 0.0)) - 1.0)


def _dlrelu(x):
    return jnp.where(jnp.abs(x) <= 10.0, x, 0.01 * x)


def _sh_from_ev(ev):
    n2 = jnp.sum(ev * ev, axis=1, keepdims=True)
    nrm = jnp.sqrt(n2)
    scale = _SQ3 / (nrm + 1e-9)
    sh = jnp.concatenate([jnp.ones_like(n2), ev[:, 0:3] * scale], axis=1)
    return sh, n2


def _ef_from_n2(n2, eemb_w, eemb_b):
    d = jnp.sqrt(n2 + 1e-12)
    dj = d - lax.broadcasted_iota(jnp.int32, (1, 10), 1).astype(jnp.float32)
    basis = jnp.exp(-dj * dj * 2.0) * (1.0 / (0.5 * math.sqrt(2.0 * math.pi)))
    return _elu(basis @ eemb_w + eemb_b)


def _full(shape):
    nd = len(shape)
    return pl.BlockSpec(shape, lambda i: (0,) * nd)


def _eblk(w):
    return pl.BlockSpec((BE, w), lambda i: (i, 0))


def _k_embed_node(h0, w, b):
    def body(h_ref, w_ref, b_ref, n16_ref, n32_ref):
        z = _elu(h_ref[...] @ w_ref[...] + b_ref[...])
        n16_ref[...] = z
        n32_ref[...] = jnp.concatenate([z, jnp.zeros((BN, 16), jnp.float32)], axis=1)

    return pl.pallas_call(
        body,
        grid=(N // BN,),
        in_specs=[pl.BlockSpec((BN, 128), lambda i: (i, 0)), _full((128, 16)), _full((1, 16))],
        out_specs=[pl.BlockSpec((BN, 16), lambda i: (i, 0)), pl.BlockSpec((BN, 32), lambda i: (i, 0))],
        out_shape=[jax.ShapeDtypeStruct((N, 16), jnp.float32),
                   jax.ShapeDtypeStruct((N, 32), jnp.float32)],
    )(h0, w, b.reshape(1, 16))


def _k_embed_edge(dfeat, p_dfeat, we, be, wr, br, wp, bp):
    def body(d_ref, pd_ref, we_r, be_r, wr_r, br_r, wp_r, bp_r, ef_r, rf_r, pf_r):
        d = d_ref[...]
        ef_r[...] = _elu(d @ we_r[...] + be_r[...])
        rf_r[...] = _elu(d @ wr_r[...] + br_r[...])
        pf_r[...] = _elu(pd_ref[...] @ wp_r[...] + bp_r[...])

    return pl.pallas_call(
        body,
        grid=(GE,),
        in_specs=[_eblk(10), _eblk(10), _full((10, 16)), _full((1, 16)),
                  _full((10, 16)), _full((1, 16)), _full((10, 16)), _full((1, 16))],
        out_specs=[_eblk(16), _eblk(16), _eblk(16)],
        out_shape=[jax.ShapeDtypeStruct((E, 16), jnp.float32)] * 3,
    )(dfeat, p_dfeat, we, be.reshape(1, 16), wr, br.reshape(1, 16), wp, bp.reshape(1, 16))


def _edge_mlp(nef, w1, b1, w2, b2):
    h = jnp.maximum(nef @ w1 + b1, 0.0)
    return _dlrelu(h @ w2 + b2)


def _k_edge_conv(layer, gs, gd, ef, ref_, pef, ev, w1, b1, w2, b2, eemb_w, eemb_b):
    M = _CONV0 if layer == 0 else _CONV1
    consts = _tp_consts(M)
    ncon = len(consts)
    gw = 16 if layer == 0 else 32
    wnum = 320 if layer == 0 else 416
    od = 32 if layer == 0 else 48
    fw = M['fw']

    def body(*refs):
        (gs_r, gd_r, ef_r, rf_r, pf_r, ev_r, w1_r, b1_r, w2_r, b2_r,
         ew_r, eb_r) = refs[:12]
        crefs = refs[12:12 + ncon]
        tp_r = refs[12 + ncon]
        ev_v = ev_r[...]
        sh, n2 = _sh_from_ev(ev_v)
        if layer == 0:
            ef_v = ef_r[...]
        else:
            ef_v = _ef_from_n2(n2, ew_r[...], eb_r[...])
        gs_v = gs_r[...]
        nef = jnp.concatenate([gs_v[:, :16], gd_r[...][:, :16], ef_v, rf_r[...], pf_r[...]], axis=1)
        w = _edge_mlp(nef, w1_r[...], b1_r[...], w2_r[...], b2_r[...])
        tp_r[...] = _tp_apply(gs_v[:, :fw], sh, w, M, [c[...] for c in crefs])

    cspecs = [_full(c.shape) for c in consts]
    return pl.pallas_call(
        body,
        grid=(GE,),
        in_specs=[_eblk(gw), _eblk(gw), _eblk(16), _eblk(16), _eblk(16), _eblk(16),
                  _full((80, 80)), _full((1, 80)), _full((80, wnum)), _full((1, wnum)),
                  _full((10, 16)), _full((1, 16))] + cspecs,
        out_specs=[_eblk(od)],
        out_shape=[jax.ShapeDtypeStruct((E, od), jnp.float32)],
    )(gs, gd, ef, ref_, pef, ev, w1, b1.reshape(1, 80), w2, b2.reshape(1, wnum),
      eemb_w, eemb_b.reshape(1, 16), *consts)[0]


def _k_edge_coord(layer, gs, gd, ef, ref_, pef, ev, w1, b1, w2, b2, eemb_w, eemb_b,
                  elin_w, elin_b, elin2_w, elin2_b, fin_w, fin_b):
    M = _COORD0 if layer == 0 else _COORD1
    consts = _tp_consts(M)
    ncon = len(consts)
    gw = 32 if layer == 0 else 48
    wnum = 20 if layer == 0 else 24
    fw = M['fw']

    def body(*refs):
        (gs_r, gd_r, ef_r, rf_r, pf_r, ev_r, w1_r, b1_r, w2_r, b2_r, ew_r, eb_r,
         el_w, el_b, el2_w, el2_b, fi_w, fi_b) = refs[:18]
        crefs = refs[18:18 + ncon]
        cu_r = refs[18 + ncon]
        efo_r = refs[19 + ncon] if layer == 1 else None
        ev_v = ev_r[...]
        sh, n2 = _sh_from_ev(ev_v)
        if layer == 0:
            ef_v = ef_r[...]
        else:
            ef_v = _ef_from_n2(n2, ew_r[...], eb_r[...])
        gs_v = gs_r[...]
        gd_v = gd_r[...]
        pair = jnp.concatenate([gs_v[:, :16], gd_v[:, :16]], axis=1)
        nef = jnp.concatenate([pair, ef_v, rf_r[...], pf_r[...]], axis=1)
        w = _edge_mlp(nef, w1_r[...], b1_r[...], w2_r[...], b2_r[...])
        cu_r[...] = _tp_apply(gs_v[:, :fw], sh, w, M, [c[...] for c in crefs])
        if layer == 1:
            z = _elu(pair @ el_w[...] + el_b[...])
            z = _elu(z @ el2_w[...] + el2_b[...])
            z = _elu(z @ fi_w[...] + fi_b[...])
            efo_r[...] = jnp.concatenate([z, jnp.zeros((BE, 7), jnp.float32)], axis=1)

    cspecs = [_full(c.shape) for c in consts]
    out_specs = [_eblk(16)]
    out_shape = [jax.ShapeDtypeStruct((E, 16), jnp.float32)]
    if layer == 1:
        out_specs.append(_eblk(8))
        out_shape.append(jax.ShapeDtypeStruct((E, 8), jnp.float32))
    res = pl.pallas_call(
        body,
        grid=(GE,),
        in_specs=[_eblk(gw), _eblk(gw), _eblk(16), _eblk(16), _eblk(16), _eblk(16),
                  _full((80, 80)), _full((1, 80)), _full((80, wnum)), _full((1, wnum)),
                  _full((10, 16)), _full((1, 16)),
                  _full((32, 16)), _full((1, 16)), _full((16, 16)), _full((1, 16)),
                  _full((16, 1)), _full((1, 1))] + cspecs,
        out_specs=out_specs,
        out_shape=out_shape,
    )(gs, gd, ef, ref_, pef, ev, w1, b1.reshape(1, 80), w2, b2.reshape(1, wnum),
      eemb_w, eemb_b.reshape(1, 16), elin_w, elin_b.reshape(1, 16),
      elin2_w, elin2_b.reshape(1, 16), fin_w, fin_b.reshape(1, 1), *consts)
    return res if layer == 1 else res[0]


def _k_coord_update(coord, cu):
    def body(c_ref, u_ref, o_ref):
        c = c_ref[...]
        o_ref[...] = c - jnp.mean(c, axis=0, keepdims=True) + _dlrelu(u_ref[...])

    return pl.pallas_call(
        body,
        grid=(1,),
        in_specs=[_full((N, 16)), _full((N, 16))],
        out_specs=[_full((N, 16))],
        out_shape=[jax.ShapeDtypeStruct((N, 16), jnp.float32)],
    )(coord, cu)[0]


def _k_final(node2, coord1, cu1):
    def body(n_ref, c_ref, u_ref, no_ref, co_ref):
        no_ref[...] = _elu(n_ref[...])
        c = c_ref[...]
        co_ref[...] = c - jnp.mean(c, axis=0, keepdims=True) + _dlrelu(u_ref[...])

    return pl.pallas_call(
        body,
        grid=(1,),
        in_specs=[_full((N, 48)), _full((N, 16)), _full((N, 16))],
        out_specs=[_full((N, 48)), _full((N, 16))],
        out_shape=[jax.ShapeDtypeStruct((N, 48), jnp.float32),
                   jax.ShapeDtypeStruct((N, 16), jnp.float32)],
    )(node2, coord1, cu1)


def _mesh():
    return plsc.VectorSubcoreMesh(core_axis_name="c", subcore_axis_name="s",
                                  num_cores=NC, num_subcores=NSUB)


def _sds(shape, dtype=jnp.float32):
    return jax.ShapeDtypeStruct(shape, dtype)


G_ROWS = 40
S_ROWS = 80
NB_INIT = 624


def _sc_gather(tables, src2, dst2, widths, with_ev):
    nt = len(tables)
    outs = []
    for t in range(nt):
        if with_ev and t == nt - 1:
            outs.append(_sds((E, widths[t])))
        else:
            outs.append(_sds((E, widths[t])))
            outs.append(_sds((E, widths[t])))
    scratch = [pltpu.VMEM((G_ROWS, 128), jnp.int32),
               pltpu.VMEM((G_ROWS, 128), jnp.int32)]
    for t in range(nt):
        scratch.append(pltpu.VMEM((128, widths[t]), jnp.float32))
        scratch.append(pltpu.VMEM((128, widths[t]), jnp.float32))
    scratch += [pltpu.SemaphoreType.DMA] * (2 * nt)

    @functools.partial(pl.kernel, out_type=outs, mesh=_mesh(), scratch_types=scratch,
                       compiler_params=pltpu.CompilerParams(use_tc_tiling_on_sc=False))
    def k(*refs):
        tab_refs = refs[:nt]
        src_r, dst_r = refs[nt], refs[nt + 1]
        out_refs = refs[nt + 2: nt + 2 + len(outs)]
        rest = refs[nt + 2 + len(outs):]
        idxs, idxd = rest[0], rest[1]
        bufs = rest[2:2 + 2 * nt]
        sems = rest[2 + 2 * nt:]

        wid = lax.axis_index("s") * NC + lax.axis_index("c")
        r0 = pl.multiple_of(wid * G_ROWS, 8)
        nr = jnp.where(wid < 31, G_ROWS, ROWS2 - 31 * G_ROWS)
        pltpu.sync_copy(src_r.at[pl.ds(r0, G_ROWS)], idxs)
        pltpu.sync_copy(dst_r.at[pl.ds(r0, G_ROWS)], idxd)

        def chunk(j):
            off = pl.multiple_of((r0 + j) * 128, 128)
            cps = []
            for t in range(nt):
                cps.append(pltpu.async_copy(tab_refs[t].at[idxs.at[j]], bufs[2 * t], sems[2 * t]))
                cps.append(pltpu.async_copy(tab_refs[t].at[idxd.at[j]], bufs[2 * t + 1], sems[2 * t + 1]))
            for cp in cps:
                cp.wait()
            oi = 0
            for t in range(nt):
                if with_ev and t == nt - 1:
                    bs, bd = bufs[2 * t], bufs[2 * t + 1]
                    for i in range(128):
                        bs[i, :] = bs[i, :] - bd[i, :]
                    pltpu.sync_copy(bs, out_refs[oi].at[pl.ds(off, 128)])
                    oi += 1
                else:
                    pltpu.sync_copy(bufs[2 * t], out_refs[oi].at[pl.ds(off, 128)])
                    pltpu.sync_copy(bufs[2 * t + 1], out_refs[oi + 1].at[pl.ds(off, 128)])
                    oi += 2

        def lbody(j, carry):
            chunk(j)
            return carry

        lax.fori_loop(0, nr, lbody, 0)

    return k(*tables, src2, dst2)


def _sc_scatter_add(rows, dst2, init, w):
    scratch = [pltpu.VMEM((S_ROWS, 128), jnp.int32),
               pltpu.VMEM((128, w), jnp.float32),
               pltpu.VMEM_SHARED((N, w), jnp.float32),
               pltpu.SemaphoreType.DMA]

    @functools.partial(pl.kernel, out_type=[_sds((N, w))], mesh=_mesh(),
                       scratch_types=scratch,
                       compiler_params=pltpu.CompilerParams(use_tc_tiling_on_sc=False))
    def k(rows_r, dst_r, init_r, out_r, idx2, rbuf, shared, sem):
        c = lax.axis_index("c")
        s = lax.axis_index("s")

        @pl.when(c == 0)
        def _():
            n0 = pl.multiple_of(s * NB_INIT, 8)
            pltpu.sync_copy(init_r.at[pl.ds(n0, NB_INIT)], shared.at[pl.ds(n0, NB_INIT)])

            @pl.when(s == NSUB - 1)
            def _():
                pltpu.sync_copy(init_r.at[pl.ds(15 * NB_INIT, N - 15 * NB_INIT)],
                                shared.at[pl.ds(15 * NB_INIT, N - 15 * NB_INIT)])

            plsc.subcore_barrier()

            r0 = pl.multiple_of(s * S_ROWS, 8)
            nr = jnp.where(s < NSUB - 1, S_ROWS, ROWS2 - 15 * S_ROWS)
            pltpu.sync_copy(dst_r.at[pl.ds(r0, S_ROWS)], idx2)

            def lbody(j, carry):
                off = pl.multiple_of((r0 + j) * 128, 128)
                pltpu.sync_copy(rows_r.at[pl.ds(off, 128)], rbuf)
                pltpu.sync_copy(rbuf, shared.at[idx2.at[j]], add=True)
                return carry

            lax.fori_loop(0, nr, lbody, 0)

            plsc.subcore_barrier()
            pltpu.sync_copy(shared.at[pl.ds(n0, NB_INIT)], out_r.at[pl.ds(n0, NB_INIT)])

            @pl.when(s == NSUB - 1)
            def _():
                pltpu.sync_copy(shared.at[pl.ds(15 * NB_INIT, N - 15 * NB_INIT)],
                                out_r.at[pl.ds(15 * NB_INIT, N - 15 * NB_INIT)])

    return k(rows, dst2, init)[0]


def kernel(h0, x0, dfeat, p_dfeat, edge_index, params):
    p = params
    src2 = jnp.pad(edge_index[0].reshape(ROWS2, 128), ((0, 30), (0, 0)))
    dst2 = jnp.pad(edge_index[1].reshape(ROWS2, 128), ((0, 30), (0, 0)))
    coord0 = jnp.pad(x0, ((0, 0), (0, 13)))

    node0, node0p32 = _k_embed_node(h0, p['emb_w'], p['emb_b'])
    ef0, ref_, pef = _k_embed_edge(dfeat, p_dfeat, p['eemb_w'], p['eemb_b'],
                                   p['reemb_w'], p['reemb_b'], p['peemb_w'], p['peemb_b'])

    gs0, gd0, ev0 = _sc_gather([node0, coord0], src2, dst2, [16, 16], with_ev=True)
    tp0 = _k_edge_conv(0, gs0, gd0, ef0, ref_, pef, ev0,
                       p['c0_fc1_w'], p['c0_fc1_b'], p['c0_fc2_w'], p['c0_fc2_b'],
                       p['eemb_w'], p['eemb_b'])
    node1 = _sc_scatter_add(tp0, dst2, node0p32, 32)
    gs1, gd1 = _sc_gather([node1], src2, dst2, [32], with_ev=False)
    cu0e = _k_edge_coord(0, gs1, gd1, ef0, ref_, pef, ev0,
                         p['x0_fc1_w'], p['x0_fc1_b'], p['x0_fc2_w'], p['x0_fc2_b'],
                         p['eemb_w'], p['eemb_b'], p['elin_w'], p['elin_b'],
                         p['elin2_w'], p['elin2_b'], p['fin_w'], p['fin_b'])
    cu0 = _sc_scatter_add(cu0e, dst2, jnp.zeros((N, 16), jnp.float32), 16)
    coord1 = _k_coord_update(coord0, cu0)

    (ev1,) = _sc_gather([coord1], src2, dst2, [16], with_ev=True)
    tp1 = _k_edge_conv(1, gs1, gd1, ef0, ref_, pef, ev1,
                       p['c1_fc1_w'], p['c1_fc1_b'], p['c1_fc2_w'], p['c1_fc2_b'],
                       p['eemb_w'], p['eemb_b'])
    node1p48 = jnp.pad(node1, ((0, 0), (0, 16)))
    node2 = _sc_scatter_add(tp1, dst2, node1p48, 48)
    gs2, gd2 = _sc_gather([node2], src2, dst2, [48], with_ev=False)
    cu1e, efo = _k_edge_coord(1, gs2, gd2, ef0, ref_, pef, ev1,
                              p['x1_fc1_w'], p['x1_fc1_b'], p['x1_fc2_w'], p['x1_fc2_b'],
                              p['eemb_w'], p['eemb_b'], p['elin_w'], p['elin_b'],
                              p['elin2_w'], p['elin2_b'], p['fin_w'], p['fin_b'])
    cu1 = _sc_scatter_add(cu1e, dst2, jnp.zeros((N, 16), jnp.float32), 16)
    node_out, coord2 = _k_final(node2, coord1, cu1)

    return node_out[:, :40], coord2[:, :3], efo[:, :1]


# speedup vs baseline: 87.9084x; 1.0386x over previous
import functools
import math

import jax
import jax.numpy as jnp
import numpy as np
from jax import lax
from jax.experimental import pallas as pl
from jax.experimental.pallas import tpu as pltpu
from jax.experimental.pallas import tpu_sc as plsc

N = 10000
E = 160000
BE = 2000
GE = E // BE
BN = 1000
ROWS2 = E // 128
W_ROWS = 39
T_ROWS = 78
NC, NSUB = 2, 16

_SQ3 = math.sqrt(3.0)
_S20 = 1.0 / math.sqrt(20.0)
_S24 = 1.0 / math.sqrt(24.0)
_S48 = 1.0 / math.sqrt(48.0)

_EPS3 = np.zeros((3, 3, 3), np.float32)
for _i, _j, _k in [(0, 1, 2), (1, 2, 0), (2, 0, 1)]:
    _EPS3[_i, _j, _k] = 1.0
    _EPS3[_i, _k, _j] = -1.0


def _build_conv0():
    Kw = 320
    A1 = np.zeros((16, Kw), np.float32)
    C1 = np.zeros((Kw, 20), np.float32)
    for u in range(16):
        for v in range(16):
            A1[u, u * 16 + v] = 1
            C1[u * 16 + v, v] = 0.25
        for v in range(4):
            A1[u, 256 + u * 4 + v] = 1
            C1[256 + u * 4 + v, 16 + v] = 0.25
    od = 32
    PDIR = np.zeros((20, od), np.float32)
    PSH = np.zeros((20, od), np.float32)
    SSH = np.zeros((4, od), np.float32)
    for v in range(16):
        PDIR[v, v] = 1
    for v in range(4):
        for j in range(3):
            PSH[16 + v, 16 + v * 3 + j] = 1
            SSH[1 + j, 16 + v * 3 + j] = 1
    return dict(A0=None, B0=None, C0=None, wsplit=[(0, 320, None)],
                A1=A1, C1=C1, PDIR=PDIR, PSH=PSH, SSH=SSH, fw=16)


def _build_conv1():
    A0 = np.zeros((28, 36), np.float32)
    B0 = np.zeros((4, 36), np.float32)
    C0 = np.zeros((36, 16), np.float32)
    for u in range(4):
        for i in range(3):
            for j in range(3):
                c = u * 9 + i * 3 + j
                A0[16 + u * 3 + i, c] = 1
                B0[1 + j, c] = 1
                for k in range(3):
                    if _EPS3[i, j, k] != 0:
                        C0[c, 4 + u * 3 + k] += _EPS3[i, j, k]
            C0[u * 9 + i * 3 + i, u] = 1
    WDE = np.zeros((32, 96), np.float32)
    for t in range(16):
        for j in range(3):
            WDE[t, t * 3 + j] = 1
            WDE[16 + t, 48 + t * 3 + j] = 1
    Kw = 480
    A1 = np.zeros((44, Kw), np.float32)
    C1 = np.zeros((Kw, 44), np.float32)
    for u in range(16):
        for v in range(16):
            A1[u, u * 16 + v] = 1
            C1[u * 16 + v, v] = _S20
        for v in range(4):
            A1[u, 320 + u * 4 + v] = 1
            C1[320 + u * 4 + v, 16 + v] = _S20
    for u in range(4):
        for v in range(16):
            A1[28 + u, 256 + u * 16 + v] = 1
            C1[256 + u * 16 + v, v] = _S20 / _SQ3
        for v in range(4):
            for j in range(3):
                A1[16 + u * 3 + j, 384 + (u * 4 + v) * 3 + j] = 1
                C1[384 + (u * 4 + v) * 3 + j, 20 + v * 3 + j] = _S20
                A1[32 + u * 3 + j, 432 + (u * 4 + v) * 3 + j] = 1
                C1[432 + (u * 4 + v) * 3 + j, 32 + v * 3 + j] = 1.0 / (2.0 * math.sqrt(2.0))
    od = 48
    PDIR = np.zeros((44, od), np.float32)
    PSH = np.zeros((44, od), np.float32)
    SSH = np.zeros((4, od), np.float32)
    for v in range(16):
        PDIR[v, v] = 1
    for v in range(4):
        for j in range(3):
            PDIR[20 + v * 3 + j, 16 + v * 3 + j] = 1
            PSH[16 + v, 16 + v * 3 + j] = 1
            SSH[1 + j, 16 + v * 3 + j] = 1
            PDIR[32 + v * 3 + j, 28 + v * 3 + j] = 1
    return dict(A0=A0, B0=B0, C0=C0, wsplit=[(0, 384, None), (384, 416, WDE)],
                A1=A1, C1=C1, PDIR=PDIR, PSH=PSH, SSH=SSH, fw=28)


def _wq():
    WQ = np.zeros((4, 12), np.float32)
    for u in range(4):
        for j in range(3):
            WQ[u, u * 3 + j] = 1
    return WQ


def _build_coord0():
    Kw = 28
    A1 = np.zeros((28, Kw), np.float32)
    C1 = np.zeros((Kw, 4), np.float32)
    for u in range(16):
        A1[u, u] = 1
        C1[u, 0] = _S20
    for u in range(4):
        for j in range(3):
            A1[16 + u * 3 + j, 16 + u * 3 + j] = 1
            C1[16 + u * 3 + j, 1 + j] = _S20
    od = 16
    PDIR = np.zeros((4, od), np.float32)
    PSH = np.zeros((4, od), np.float32)
    SSH = np.zeros((4, od), np.float32)
    for j in range(3):
        PDIR[1 + j, 1 + j] = 1
        PSH[0, 1 + j] = 1
        SSH[1 + j, 1 + j] = 1
    return dict(A0=None, B0=None, C0=None, wsplit=[(0, 16, None), (16, 20, _wq())],
                A1=A1, C1=C1, PDIR=PDIR, PSH=PSH, SSH=SSH, fw=28)


def _build_coord1():
    pairs = [(i, j) for i in range(3) for j in range(3) if i != j]
    A0 = np.zeros((40, 24), np.float32)
    B0 = np.zeros((4, 24), np.float32)
    C0 = np.zeros((24, 12), np.float32)
    for u in range(4):
        for pi, (i, j) in enumerate(pairs):
            c = u * 6 + pi
            A0[28 + u * 3 + i, c] = 1
            B0[1 + j, c] = 1
            for k in range(3):
                if _EPS3[i, j, k] != 0:
                    C0[c, u * 3 + k] += _EPS3[i, j, k]
    Kw = 40
    A1 = np.zeros((52, Kw), np.float32)
    C1 = np.zeros((Kw, 7), np.float32)
    for u in range(16):
        A1[u, u] = 1
        C1[u, 0] = _S24
    for u in range(4):
        for j in range(3):
            A1[16 + u * 3 + j, 16 + u * 3 + j] = 1
            C1[16 + u * 3 + j, 1 + j] = _S24
            A1[40 + u * 3 + j, 28 + u * 3 + j] = 1
            C1[28 + u * 3 + j, 4 + j] = _S48
    od = 16
    PDIR = np.zeros((7, od), np.float32)
    PSH = np.zeros((7, od), np.float32)
    SSH = np.zeros((4, od), np.float32)
    for j in range(3):
        PDIR[1 + j, 1 + j] = 1
        PDIR[4 + j, 1 + j] = 1
        PSH[0, 1 + j] = 1
        SSH[1 + j, 1 + j] = 1
    return dict(A0=A0, B0=B0, C0=C0,
                wsplit=[(0, 16, None), (16, 20, _wq()), (20, 24, _wq())],
                A1=A1, C1=C1, PDIR=PDIR, PSH=PSH, SSH=SSH, fw=40)


_CONV0 = _build_conv0()
_CONV1 = _build_conv1()
_COORD0 = _build_coord0()
_COORD1 = _build_coord1()


def _tp_consts(M):
    out = []
    if M['A0'] is not None:
        out += [jnp.asarray(M['A0']), jnp.asarray(M['B0']), jnp.asarray(M['C0'])]
    for (_, _, mat) in M['wsplit']:
        if mat is not None:
            out.append(jnp.asarray(mat))
    out += [jnp.asarray(M['A1']), jnp.asarray(M['C1']), jnp.asarray(M['PDIR']),
            jnp.asarray(M['PSH']), jnp.asarray(M['SSH'])]
    return out


def _tp_apply(feat, sh, w, M, consts):
    it = iter(consts)
    if M['A0'] is not None:
        A0, B0, C0 = next(it), next(it), next(it)
        aux = ((feat @ A0) * (sh @ B0)) @ C0
        feat2 = jnp.concatenate([feat, aux], axis=1)
    else:
        feat2 = feat
    parts = []
    for (a, b, mat) in M['wsplit']:
        if mat is None:
            parts.append(w[:, a:b])
        else:
            parts.append(w[:, a:b] @ next(it))
    wext = jnp.concatenate(parts, axis=1) if len(parts) > 1 else parts[0]
    A1, C1, PDIR, PSH, SSH = next(it), next(it), next(it), next(it), next(it)
    part = ((feat2 @ A1) * wext) @ C1
    return part @ PDIR + (part @ PSH) * (sh @ SSH)


def _elu(x):
    return jnp.where(x > 0, x, jnp.exp(jnp.minimum(x, 0.0)) - 1.0)


def _dlrelu(x):
    return jnp.where(jnp.abs(x) <= 10.0, x, 0.01 * x)


_GBD = np.zeros((128, 128), np.float32)
for _i in range(128):
    for _j in range(128):
        if _i // 16 == _j // 16:
            _GBD[_i, _j] = 1.0
_GM123 = np.zeros((1, 128), np.float32)
_GONE0 = np.zeros((1, 128), np.float32)
_GM10 = np.zeros((1, 128), np.float32)
_GJC = np.zeros((1, 128), np.float32)
for _g in range(8):
    _GONE0[0, _g * 16] = 1.0
    for _c in range(3):
        _GM123[0, _g * 16 + 1 + _c] = 1.0
    for _j in range(10):
        _GM10[0, _g * 16 + _j] = 1.0
        _GJC[0, _g * 16 + _j] = float(_j)

BG = 2000


def _k_geom(evp, eemb128, ebias128, with_ef):
    consts = [jnp.asarray(_GBD), jnp.asarray(_GM123), jnp.asarray(_GONE0),
              jnp.asarray(_GM10), jnp.asarray(_GJC)]

    def body(ev_r, bd_r, m123_r, one0_r, m10_r, jc_r, e128_r, eb_r, sh_r, ef_r):
        ev = ev_r[...]
        n2g = (ev * ev) @ bd_r[...]
        scale = _SQ3 / (jnp.sqrt(n2g) + 1e-9)
        sh_r[...] = ev * scale * m123_r[...] + one0_r[...]
        if with_ef:
            d = jnp.sqrt(n2g + 1e-12)
            dj = d - jc_r[...]
            basis = jnp.exp(dj * dj * (-2.0)) * (m10_r[...] * (1.0 / (0.5 * math.sqrt(2.0 * math.pi))))
            ef_r[...] = _elu(basis @ e128_r[...] + eb_r[...])
        else:
            ef_r[...] = jnp.zeros_like(ev)

    rows = E // 8
    blk = pl.BlockSpec((BG, 128), lambda i: (i, 0))
    res = pl.pallas_call(
        body,
        grid=(rows // BG,),
        in_specs=[blk, _full((128, 128)), _full((1, 128)), _full((1, 128)),
                  _full((1, 128)), _full((1, 128)), _full((128, 128)), _full((1, 128))],
        out_specs=[blk, blk],
        out_shape=[jax.ShapeDtypeStruct((rows, 128), jnp.float32)] * 2,
    )(evp, *consts, eemb128, ebias128)
    sh = res[0].reshape(E, 16)
    ef = res[1].reshape(E, 16)
    return sh, ef


def _full(shape):
    nd = len(shape)
    return pl.BlockSpec(shape, lambda i: (0,) * nd)


def _eblk(w):
    return pl.BlockSpec((BE, w), lambda i: (i, 0))


def _k_embed_node(h0, w, b):
    def body(h_ref, w_ref, b_ref, n16_ref, n32_ref):
        z = _elu(h_ref[...] @ w_ref[...] + b_ref[...])
        n16_ref[...] = z
        n32_ref[...] = jnp.concatenate([z, jnp.zeros((BN, 16), jnp.float32)], axis=1)

    return pl.pallas_call(
        body,
        grid=(N // BN,),
        in_specs=[pl.BlockSpec((BN, 128), lambda i: (i, 0)), _full((128, 16)), _full((1, 16))],
        out_specs=[pl.BlockSpec((BN, 16), lambda i: (i, 0)), pl.BlockSpec((BN, 32), lambda i: (i, 0))],
        out_shape=[jax.ShapeDtypeStruct((N, 16), jnp.float32),
                   jax.ShapeDtypeStruct((N, 32), jnp.float32)],
    )(h0, w, b.reshape(1, 16))


def _k_embed_edge(dfeat, p_dfeat, we, be, wr, br, wp, bp):
    def body(d_ref, pd_ref, we_r, be_r, wr_r, br_r, wp_r, bp_r, ef_r, rf_r, pf_r):
        d = d_ref[...]
        ef_r[...] = _elu(d @ we_r[...] + be_r[...])
        rf_r[...] = _elu(d @ wr_r[...] + br_r[...])
        pf_r[...] = _elu(pd_ref[...] @ wp_r[...] + bp_r[...])

    return pl.pallas_call(
        body,
        grid=(GE,),
        in_specs=[_eblk(10), _eblk(10), _full((10, 16)), _full((1, 16)),
                  _full((10, 16)), _full((1, 16)), _full((10, 16)), _full((1, 16))],
        out_specs=[_eblk(16), _eblk(16), _eblk(16)],
        out_shape=[jax.ShapeDtypeStruct((E, 16), jnp.float32)] * 3,
    )(dfeat, p_dfeat, we, be.reshape(1, 16), wr, br.reshape(1, 16), wp, bp.reshape(1, 16))


def _edge_mlp(nef, w1, b1, w2, b2):
    h = jnp.maximum(nef @ w1 + b1, 0.0)
    return _dlrelu(h @ w2 + b2)


def _k_edge_conv(layer, gs, gd, ef, ref_, pef, sh_e, w1, b1, w2, b2):
    M = _CONV0 if layer == 0 else _CONV1
    consts = _tp_consts(M)
    ncon = len(consts)
    gw = 16 if layer == 0 else 32
    wnum = 320 if layer == 0 else 416
    od = 32 if layer == 0 else 48
    fw = M['fw']

    def body(*refs):
        (gs_r, gd_r, ef_r, rf_r, pf_r, sh_r, w1_r, b1_r, w2_r, b2_r) = refs[:10]
        crefs = refs[10:10 + ncon]
        tp_r = refs[10 + ncon]
        sh = sh_r[...][:, :4]
        gs_v = gs_r[...]
        nef = jnp.concatenate([gs_v[:, :16], gd_r[...][:, :16], ef_r[...], rf_r[...], pf_r[...]], axis=1)
        w = _edge_mlp(nef, w1_r[...], b1_r[...], w2_r[...], b2_r[...])
        tp_r[...] = _tp_apply(gs_v[:, :fw], sh, w, M, [c[...] for c in crefs])

    cspecs = [_full(c.shape) for c in consts]
    return pl.pallas_call(
        body,
        grid=(GE,),
        in_specs=[_eblk(gw), _eblk(gw), _eblk(16), _eblk(16), _eblk(16), _eblk(16),
                  _full((80, 80)), _full((1, 80)), _full((80, wnum)), _full((1, wnum))] + cspecs,
        out_specs=[_eblk(od)],
        out_shape=[jax.ShapeDtypeStruct((E, od), jnp.float32)],
    )(gs, gd, ef, ref_, pef, sh_e, w1, b1.reshape(1, 80), w2, b2.reshape(1, wnum),
      *consts)[0]


def _k_edge_coord(layer, gs, gd, ef, ref_, pef, sh_e, w1, b1, w2, b2,
                  elin_w, elin_b, elin2_w, elin2_b, fin_w, fin_b):
    M = _COORD0 if layer == 0 else _COORD1
    consts = _tp_consts(M)
    ncon = len(consts)
    gw = 32 if layer == 0 else 48
    wnum = 20 if layer == 0 else 24
    fw = M['fw']

    def body(*refs):
        (gs_r, gd_r, ef_r, rf_r, pf_r, sh_r, w1_r, b1_r, w2_r, b2_r,
         el_w, el_b, el2_w, el2_b, fi_w, fi_b) = refs[:16]
        crefs = refs[16:16 + ncon]
        cu_r = refs[16 + ncon]
        efo_r = refs[17 + ncon] if layer == 1 else None
        sh = sh_r[...][:, :4]
        gs_v = gs_r[...]
        gd_v = gd_r[...]
        pair = jnp.concatenate([gs_v[:, :16], gd_v[:, :16]], axis=1)
        nef = jnp.concatenate([pair, ef_r[...], rf_r[...], pf_r[...]], axis=1)
        w = _edge_mlp(nef, w1_r[...], b1_r[...], w2_r[...], b2_r[...])
        cu_r[...] = _tp_apply(gs_v[:, :fw], sh, w, M, [c[...] for c in crefs])
        if layer == 1:
            z = _elu(pair @ el_w[...] + el_b[...])
            z = _elu(z @ el2_w[...] + el2_b[...])
            z = _elu(z @ fi_w[...] + fi_b[...])
            efo_r[...] = jnp.concatenate([z, jnp.zeros((BE, 7), jnp.float32)], axis=1)

    cspecs = [_full(c.shape) for c in consts]
    out_specs = [_eblk(16)]
    out_shape = [jax.ShapeDtypeStruct((E, 16), jnp.float32)]
    if layer == 1:
        out_specs.append(_eblk(8))
        out_shape.append(jax.ShapeDtypeStruct((E, 8), jnp.float32))
    res = pl.pallas_call(
        body,
        grid=(GE,),
        in_specs=[_eblk(gw), _eblk(gw), _eblk(16), _eblk(16), _eblk(16), _eblk(16),
                  _full((80, 80)), _full((1, 80)), _full((80, wnum)), _full((1, wnum)),
                  _full((32, 16)), _full((1, 16)), _full((16, 16)), _full((1, 16)),
                  _full((16, 1)), _full((1, 1))] + cspecs,
        out_specs=out_specs,
        out_shape=out_shape,
    )(gs, gd, ef, ref_, pef, sh_e, w1, b1.reshape(1, 80), w2, b2.reshape(1, wnum),
      elin_w, elin_b.reshape(1, 16), elin2_w, elin2_b.reshape(1, 16),
      fin_w, fin_b.reshape(1, 1), *consts)
    return res if layer == 1 else res[0]


def _k_coord_update(coord, cu):
    def body(c_ref, u_ref, o_ref):
        c = c_ref[...]
        o_ref[...] = c - jnp.mean(c, axis=0, keepdims=True) + _dlrelu(u_ref[...])

    return pl.pallas_call(
        body,
        grid=(1,),
        in_specs=[_full((N, 16)), _full((N, 16))],
        out_specs=[_full((N, 16))],
        out_shape=[jax.ShapeDtypeStruct((N, 16), jnp.float32)],
    )(coord, cu)[0]


def _k_final(node2, coord1, cu1):
    def body(n_ref, c_ref, u_ref, no_ref, co_ref):
        no_ref[...] = _elu(n_ref[...])
        c = c_ref[...]
        co_ref[...] = c - jnp.mean(c, axis=0, keepdims=True) + _dlrelu(u_ref[...])

    return pl.pallas_call(
        body,
        grid=(1,),
        in_specs=[_full((N, 48)), _full((N, 16)), _full((N, 16))],
        out_specs=[_full((N, 48)), _full((N, 16))],
        out_shape=[jax.ShapeDtypeStruct((N, 48), jnp.float32),
                   jax.ShapeDtypeStruct((N, 16), jnp.float32)],
    )(node2, coord1, cu1)


def _mesh():
    return plsc.VectorSubcoreMesh(core_axis_name="c", subcore_axis_name="s",
                                  num_cores=NC, num_subcores=NSUB)


def _sds(shape, dtype=jnp.float32):
    return jax.ShapeDtypeStruct(shape, dtype)


G_ROWS = 40
S_ROWS = 80
NB_INIT = 624


def _sc_gather(tables, src2, dst2, widths, with_ev):
    nt = len(tables)
    outs = []
    for t in range(nt):
        if with_ev and t == nt - 1:
            outs.append(_sds((E, widths[t])))
        else:
            outs.append(_sds((E, widths[t])))
            outs.append(_sds((E, widths[t])))
    scratch = [pltpu.VMEM((G_ROWS, 128), jnp.int32),
               pltpu.VMEM((G_ROWS, 128), jnp.int32)]
    for t in range(nt):
        scratch.append(pltpu.VMEM((128, widths[t]), jnp.float32))
        scratch.append(pltpu.VMEM((128, widths[t]), jnp.float32))
    scratch += [pltpu.SemaphoreType.DMA] * (2 * nt)

    @functools.partial(pl.kernel, out_type=outs, mesh=_mesh(), scratch_types=scratch,
                       compiler_params=pltpu.CompilerParams(use_tc_tiling_on_sc=False))
    def k(*refs):
        tab_refs = refs[:nt]
        src_r, dst_r = refs[nt], refs[nt + 1]
        out_refs = refs[nt + 2: nt + 2 + len(outs)]
        rest = refs[nt + 2 + len(outs):]
        idxs, idxd = rest[0], rest[1]
        bufs = rest[2:2 + 2 * nt]
        sems = rest[2 + 2 * nt:]

        wid = lax.axis_index("s") * NC + lax.axis_index("c")
        r0 = pl.multiple_of(wid * G_ROWS, 8)
        nr = jnp.where(wid < 31, G_ROWS, ROWS2 - 31 * G_ROWS)
        pltpu.sync_copy(src_r.at[pl.ds(r0, G_ROWS)], idxs)
        pltpu.sync_copy(dst_r.at[pl.ds(r0, G_ROWS)], idxd)

        def chunk(j):
            off = pl.multiple_of((r0 + j) * 128, 128)
            cps = []
            for t in range(nt):
                cps.append(pltpu.async_copy(tab_refs[t].at[idxs.at[j]], bufs[2 * t], sems[2 * t]))
                cps.append(pltpu.async_copy(tab_refs[t].at[idxd.at[j]], bufs[2 * t + 1], sems[2 * t + 1]))
            for cp in cps:
                cp.wait()
            oi = 0
            for t in range(nt):
                if with_ev and t == nt - 1:
                    bs, bd = bufs[2 * t], bufs[2 * t + 1]
                    for i in range(128):
                        bs[i, :] = bs[i, :] - bd[i, :]
                    pltpu.sync_copy(bs, out_refs[oi].at[pl.ds(off, 128)])
                    oi += 1
                else:
                    pltpu.sync_copy(bufs[2 * t], out_refs[oi].at[pl.ds(off, 128)])
                    pltpu.sync_copy(bufs[2 * t + 1], out_refs[oi + 1].at[pl.ds(off, 128)])
                    oi += 2

        def lbody(j, carry):
            chunk(j)
            return carry

        lax.fori_loop(0, nr, lbody, 0)

    return k(*tables, src2, dst2)


def _sc_scatter_add(rows, dst2, init, w):
    scratch = [pltpu.VMEM((S_ROWS, 128), jnp.int32),
               pltpu.VMEM((128, w), jnp.float32),
               pltpu.VMEM_SHARED((N, w), jnp.float32),
               pltpu.SemaphoreType.DMA]

    @functools.partial(pl.kernel, out_type=[_sds((N, w))], mesh=_mesh(),
                       scratch_types=scratch,
                       compiler_params=pltpu.CompilerParams(use_tc_tiling_on_sc=False))
    def k(rows_r, dst_r, init_r, out_r, idx2, rbuf, shared, sem):
        c = lax.axis_index("c")
        s = lax.axis_index("s")

        @pl.when(c == 0)
        def _():
            n0 = pl.multiple_of(s * NB_INIT, 8)
            pltpu.sync_copy(init_r.at[pl.ds(n0, NB_INIT)], shared.at[pl.ds(n0, NB_INIT)])

            @pl.when(s == NSUB - 1)
            def _():
                pltpu.sync_copy(init_r.at[pl.ds(15 * NB_INIT, N - 15 * NB_INIT)],
                                shared.at[pl.ds(15 * NB_INIT, N - 15 * NB_INIT)])

            plsc.subcore_barrier()

            r0 = pl.multiple_of(s * S_ROWS, 8)
            nr = jnp.where(s < NSUB - 1, S_ROWS, ROWS2 - 15 * S_ROWS)
            pltpu.sync_copy(dst_r.at[pl.ds(r0, S_ROWS)], idx2)

            def lbody(j, carry):
                off = pl.multiple_of((r0 + j) * 128, 128)
                pltpu.sync_copy(rows_r.at[pl.ds(off, 128)], rbuf)
                pltpu.sync_copy(rbuf, shared.at[idx2.at[j]], add=True)
                return carry

            lax.fori_loop(0, nr, lbody, 0)

            plsc.subcore_barrier()
            pltpu.sync_copy(shared.at[pl.ds(n0, NB_INIT)], out_r.at[pl.ds(n0, NB_INIT)])

            @pl.when(s == NSUB - 1)
            def _():
                pltpu.sync_copy(shared.at[pl.ds(15 * NB_INIT, N - 15 * NB_INIT)],
                                out_r.at[pl.ds(15 * NB_INIT, N - 15 * NB_INIT)])

    return k(rows, dst2, init)[0]


def kernel(h0, x0, dfeat, p_dfeat, edge_index, params):
    p = params
    src2 = jnp.pad(edge_index[0].reshape(ROWS2, 128), ((0, 30), (0, 0)))
    dst2 = jnp.pad(edge_index[1].reshape(ROWS2, 128), ((0, 30), (0, 0)))
    coord0 = jnp.pad(x0, ((0, 0), (1, 12)))
    eemb128 = jnp.kron(jnp.eye(8, dtype=jnp.float32),
                       jnp.pad(p['eemb_w'], ((0, 6), (0, 0))))
    ebias128 = jnp.tile(p['eemb_b'], 8).reshape(1, 128)

    node0, node0p32 = _k_embed_node(h0, p['emb_w'], p['emb_b'])
    ef0, ref_, pef = _k_embed_edge(dfeat, p_dfeat, p['eemb_w'], p['eemb_b'],
                                   p['reemb_w'], p['reemb_b'], p['peemb_w'], p['peemb_b'])

    gs0, gd0, ev0 = _sc_gather([node0, coord0], src2, dst2, [16, 16], with_ev=True)
    sh0, _ = _k_geom(ev0.reshape(E // 8, 128), eemb128, ebias128, with_ef=False)
    tp0 = _k_edge_conv(0, gs0, gd0, ef0, ref_, pef, sh0,
                       p['c0_fc1_w'], p['c0_fc1_b'], p['c0_fc2_w'], p['c0_fc2_b'])
    node1 = _sc_scatter_add(tp0, dst2, node0p32, 32)
    gs1, gd1 = _sc_gather([node1], src2, dst2, [32], with_ev=False)
    cu0e = _k_edge_coord(0, gs1, gd1, ef0, ref_, pef, sh0,
                         p['x0_fc1_w'], p['x0_fc1_b'], p['x0_fc2_w'], p['x0_fc2_b'],
                         p['elin_w'], p['elin_b'],
                         p['elin2_w'], p['elin2_b'], p['fin_w'], p['fin_b'])
    cu0 = _sc_scatter_add(cu0e, dst2, jnp.zeros((N, 16), jnp.float32), 16)
    coord1 = _k_coord_update(coord0, cu0)

    (ev1,) = _sc_gather([coord1], src2, dst2, [16], with_ev=True)
    sh1, ef1 = _k_geom(ev1.reshape(E // 8, 128), eemb128, ebias128, with_ef=True)
    tp1 = _k_edge_conv(1, gs1, gd1, ef1, ref_, pef, sh1,
                       p['c1_fc1_w'], p['c1_fc1_b'], p['c1_fc2_w'], p['c1_fc2_b'])
    node1p48 = jnp.pad(node1, ((0, 0), (0, 16)))
    node2 = _sc_scatter_add(tp1, dst2, node1p48, 48)
    gs2, gd2 = _sc_gather([node2], src2, dst2, [48], with_ev=False)
    cu1e, efo = _k_edge_coord(1, gs2, gd2, ef1, ref_, pef, sh1,
                              p['x1_fc1_w'], p['x1_fc1_b'], p['x1_fc2_w'], p['x1_fc2_b'],
                              p['elin_w'], p['elin_b'],
                              p['elin2_w'], p['elin2_b'], p['fin_w'], p['fin_b'])
    cu1 = _sc_scatter_add(cu1e, dst2, jnp.zeros((N, 16), jnp.float32), 16)
    node_out, coord2 = _k_final(node2, coord1, cu1)

    return node_out[:, :40], coord2[:, 1:4], efo[:, :1]
